# speedup vs baseline: 1.8992x; 1.8992x over previous
import jax
import jax.numpy as jnp
from jax import lax
from jax.experimental import pallas as pl
from jax.experimental.pallas import tpu as pltpu

_BN = 1000


def _fused_body(mb_ref, ap_ref, cl_ref, ha_ref, pc_ref, comp_ref):
    i = pl.program_id(1)
    bn = ap_ref.shape[1]
    k = ap_ref.shape[2]
    c = cl_ref.shape[2]

    probs = ap_ref[0]
    iota_k = lax.broadcasted_iota(jnp.int32, (bn, k), 1)
    pmax = jnp.max(probs, axis=1, keepdims=True)
    ha = jnp.min(jnp.where(probs == pmax, iota_k, k), axis=1, keepdims=True)
    ha_ref[0] = ha

    logits = cl_ref[0]
    iota_c = lax.broadcasted_iota(jnp.int32, (bn, c), 1)
    lmax = jnp.max(logits, axis=1, keepdims=True)
    pc_ref[0] = jnp.min(jnp.where(logits == lmax, iota_c, c), axis=1, keepdims=True)

    mask = iota_k == ha
    mb = mb_ref[0]

    @pl.when(i == 0)
    def _():
        row = lax.broadcasted_iota(jnp.int32, (4, k), 0)
        comp_ref[0] = jnp.where(row < 2, 1.0, 0.0).astype(jnp.float32)

    contribs = []
    for cc in range(4):
        s = 1.0 if cc < 2 else -1.0
        fill = 1.0 if cc < 2 else 0.0
        v = mb[:, cc : cc + 1] * s
        contribs.append(jnp.min(jnp.where(mask, v, fill), axis=0, keepdims=True))
    comp_ref[0] = jnp.minimum(comp_ref[0], jnp.concatenate(contribs, axis=0))


def kernel(micro_boxes, assign_probs, class_logits):
    b, n, _ = micro_boxes.shape
    k = assign_probs.shape[-1]
    c = class_logits.shape[-1]
    nblk = n // _BN

    ha, pc, comp = pl.pallas_call(
        _fused_body,
        grid=(b, nblk),
        in_specs=[
            pl.BlockSpec((1, _BN, 4), lambda bb, ii: (bb, ii, 0)),
            pl.BlockSpec((1, _BN, k), lambda bb, ii: (bb, ii, 0)),
            pl.BlockSpec((1, _BN, c), lambda bb, ii: (bb, ii, 0)),
        ],
        out_specs=[
            pl.BlockSpec((1, _BN, 1), lambda bb, ii: (bb, ii, 0)),
            pl.BlockSpec((1, _BN, 1), lambda bb, ii: (bb, ii, 0)),
            pl.BlockSpec((1, 4, k), lambda bb, ii: (bb, 0, 0)),
        ],
        out_shape=[
            jax.ShapeDtypeStruct((b, n, 1), jnp.int32),
            jax.ShapeDtypeStruct((b, n, 1), jnp.int32),
            jax.ShapeDtypeStruct((b, 4, k), jnp.float32),
        ],
        compiler_params=pltpu.CompilerParams(
            dimension_semantics=("parallel", "arbitrary"),
        ),
    )(micro_boxes, assign_probs, class_logits)

    hard_assign = ha.reshape(b, n)
    pred_classes = pc.reshape(b, n)
    signs = jnp.array([1.0, 1.0, -1.0, -1.0], jnp.float32)
    comp_boxes = jnp.transpose(comp * signs[None, :, None], (0, 2, 1))
    micro_keep_mask = jnp.ones((b, n), dtype=bool)
    component_ids = jnp.broadcast_to(jnp.arange(k, dtype=jnp.int32), (b, k))
    return (hard_assign, pred_classes, micro_boxes, micro_keep_mask, comp_boxes, component_ids)
